# baseline (speedup 1.0000x reference)
import jax
import jax.numpy as jnp
import numpy as np
from jax.experimental import pallas as pl
from jax.experimental.pallas import tpu as pltpu

N = 16777216
BM = 512

C_T = np.float32(np.float64(16) / (2.0 * np.pi))
C_P = np.float32(np.float64(8) / np.pi)


def _tc_body(grid_ref, t_ref, p_ref, o_ref):
    tab = jnp.exp(jnp.clip(grid_ref[...], -0.3, 0.3))
    t = t_ref[...]
    p = p_ref[...]
    ti = (t * C_T).astype(jnp.int32)
    pi = (p * C_P).astype(jnp.int32)
    ti = jnp.minimum(ti, 15)
    pi = jnp.minimum(pi, 7)
    idx = ti * 8 + pi
    tab_b = jnp.broadcast_to(tab[:1], (BM, 128))
    o_ref[...] = jnp.take_along_axis(tab_b, idx, axis=1)


@jax.jit
def _tc_call(theta, phi, gridf):
    rows = N // 128
    t2 = theta.reshape(rows, 128)
    p2 = phi.reshape(rows, 128)
    g2 = jnp.broadcast_to(gridf.reshape(1, 128), (8, 128))
    out = pl.pallas_call(
        _tc_body,
        grid=(rows // BM,),
        in_specs=[
            pl.BlockSpec((8, 128), lambda i: (0, 0)),
            pl.BlockSpec((BM, 128), lambda i: (i, 0)),
            pl.BlockSpec((BM, 128), lambda i: (i, 0)),
        ],
        out_specs=pl.BlockSpec((BM, 128), lambda i: (i, 0)),
        out_shape=jax.ShapeDtypeStruct((rows, 128), jnp.float32),
    )(g2, t2, p2)
    return out.reshape(-1)


def kernel(theta, phi, grid):
    return _tc_call(theta, phi, grid.reshape(-1))


# speedup vs baseline: 1.9186x; 1.9186x over previous
import functools

import jax
import jax.numpy as jnp
import numpy as np
from jax import lax
from jax.experimental import pallas as pl
from jax.experimental.pallas import tpu as pltpu
from jax.experimental.pallas import tpu_sc as plsc

THETA_RES = 16
PHI_RES = 8
MAX_SCALE_LOG = 0.3
N = 16777216

NC = 2
NS = 16
NW = NC * NS
B_PER_W = N // NW
CHUNK = 16384
NCHUNK = B_PER_W // CHUNK
LANES = 16
NBUF = 2

C_T = np.float32(np.float64(THETA_RES) / (2.0 * np.pi))
C_P = np.float32(np.float64(PHI_RES) / np.pi)


def _sc_body(theta_hbm, phi_hbm, grid_hbm, out_hbm,
             tbuf0, tbuf1, pbuf0, pbuf1, obuf0, obuf1, gbuf, table,
             lsem0, lsem1, osem0, osem1):
    wid = lax.axis_index("s") * NC + lax.axis_index("c")
    base = wid * B_PER_W
    tbufs = (tbuf0, tbuf1)
    pbufs = (pbuf0, pbuf1)
    obufs = (obuf0, obuf1)
    lsems = (lsem0, lsem1)
    osems = (osem0, osem1)

    pltpu.sync_copy(grid_hbm, gbuf)

    @pl.loop(0, 8)
    def _table(v):
        g = gbuf[pl.ds(v * LANES, LANES)]
        table[pl.ds(v * LANES, LANES)] = jnp.exp(
            jnp.clip(g, -MAX_SCALE_LOG, MAX_SCALE_LOG))

    def start_load(g, b):
        off = base + g * CHUNK
        pltpu.async_copy(theta_hbm.at[pl.ds(off, CHUNK)], tbufs[b], lsems[b])
        pltpu.async_copy(phi_hbm.at[pl.ds(off, CHUNK)], pbufs[b], lsems[b])

    def wait_load(b):
        pltpu.make_async_copy(
            theta_hbm.at[pl.ds(0, CHUNK)], tbufs[b], lsems[b]).wait()
        pltpu.make_async_copy(
            phi_hbm.at[pl.ds(0, CHUNK)], pbufs[b], lsems[b]).wait()

    def start_store(g, b):
        off = base + g * CHUNK
        pltpu.async_copy(obufs[b], out_hbm.at[pl.ds(off, CHUNK)], osems[b])

    def wait_store(b):
        pltpu.make_async_copy(
            obufs[b], out_hbm.at[pl.ds(0, CHUNK)], osems[b]).wait()

    start_load(0, 0)
    start_load(1, 1)

    @pl.loop(0, NCHUNK // NBUF)
    def _step(s):
        for b in range(NBUF):
            g = s * NBUF + b
            wait_load(b)

            @pl.when(s > 0)
            def _():
                wait_store(b)

            tb = tbufs[b]
            pb = pbufs[b]
            ob = obufs[b]

            @plsc.parallel_loop(0, CHUNK // LANES, unroll=16)
            def _vec(i):
                sl = pl.ds(i * LANES, LANES)
                t = tb[sl]
                p = pb[sl]
                tn = jnp.minimum(t * C_T, np.float32(THETA_RES - 1))
                pn = jnp.minimum(p * C_P, np.float32(PHI_RES - 1))
                idx = tn.astype(jnp.int32) * PHI_RES + pn.astype(jnp.int32)
                ob[sl] = plsc.load_gather(table, [idx])

            start_store(g, b)

            @pl.when(g + NBUF < NCHUNK)
            def _():
                start_load(g + NBUF, b)

    for b in range(NBUF):
        wait_store(b)


@jax.jit
def _sc_call(theta, phi, gridf):
    mesh = plsc.VectorSubcoreMesh(core_axis_name="c", subcore_axis_name="s")
    return pl.kernel(
        _sc_body,
        out_type=jax.ShapeDtypeStruct((N,), jnp.float32),
        mesh=mesh,
        scratch_types=[
            pltpu.VMEM((CHUNK,), jnp.float32),
            pltpu.VMEM((CHUNK,), jnp.float32),
            pltpu.VMEM((CHUNK,), jnp.float32),
            pltpu.VMEM((CHUNK,), jnp.float32),
            pltpu.VMEM((CHUNK,), jnp.float32),
            pltpu.VMEM((CHUNK,), jnp.float32),
            pltpu.VMEM((THETA_RES * PHI_RES,), jnp.float32),
            pltpu.VMEM((THETA_RES * PHI_RES,), jnp.float32),
            pltpu.SemaphoreType.DMA,
            pltpu.SemaphoreType.DMA,
            pltpu.SemaphoreType.DMA,
            pltpu.SemaphoreType.DMA,
        ],
        compiler_params=pltpu.CompilerParams(needs_layout_passes=False),
    )(theta, phi, gridf)


def kernel(theta, phi, grid):
    return _sc_call(theta, phi, grid.reshape(-1))


# speedup vs baseline: 1.9198x; 1.0006x over previous
import functools

import jax
import jax.numpy as jnp
import numpy as np
from jax import lax
from jax.experimental import pallas as pl
from jax.experimental.pallas import tpu as pltpu
from jax.experimental.pallas import tpu_sc as plsc

THETA_RES = 16
PHI_RES = 8
MAX_SCALE_LOG = 0.3
N = 16777216

NC = 2
NS = 16
NW = NC * NS
B_PER_W = N // NW
CHUNK = 8192
NCHUNK = B_PER_W // CHUNK
LANES = 16
NBUF = 4

C_T = np.float32(np.float64(THETA_RES) / (2.0 * np.pi))
C_P = np.float32(np.float64(PHI_RES) / np.pi)


def _sc_body(theta_hbm, phi_hbm, grid_hbm, out_hbm,
             tbuf0, tbuf1, tbuf2, tbuf3,
             pbuf0, pbuf1, pbuf2, pbuf3,
             obuf0, obuf1, obuf2, obuf3,
             gbuf, table,
             lsem0, lsem1, lsem2, lsem3,
             osem0, osem1, osem2, osem3):
    wid = lax.axis_index("s") * NC + lax.axis_index("c")
    base = wid * B_PER_W
    tbufs = (tbuf0, tbuf1, tbuf2, tbuf3)
    pbufs = (pbuf0, pbuf1, pbuf2, pbuf3)
    obufs = (obuf0, obuf1, obuf2, obuf3)
    lsems = (lsem0, lsem1, lsem2, lsem3)
    osems = (osem0, osem1, osem2, osem3)

    pltpu.sync_copy(grid_hbm, gbuf)

    @pl.loop(0, 8)
    def _table(v):
        g = gbuf[pl.ds(v * LANES, LANES)]
        table[pl.ds(v * LANES, LANES)] = jnp.exp(
            jnp.clip(g, -MAX_SCALE_LOG, MAX_SCALE_LOG))

    def start_load(g, b):
        off = base + g * CHUNK
        pltpu.async_copy(theta_hbm.at[pl.ds(off, CHUNK)], tbufs[b], lsems[b])
        pltpu.async_copy(phi_hbm.at[pl.ds(off, CHUNK)], pbufs[b], lsems[b])

    def wait_load(b):
        pltpu.make_async_copy(
            theta_hbm.at[pl.ds(0, CHUNK)], tbufs[b], lsems[b]).wait()
        pltpu.make_async_copy(
            phi_hbm.at[pl.ds(0, CHUNK)], pbufs[b], lsems[b]).wait()

    def start_store(g, b):
        off = base + g * CHUNK
        pltpu.async_copy(obufs[b], out_hbm.at[pl.ds(off, CHUNK)], osems[b])

    def wait_store(b):
        pltpu.make_async_copy(
            obufs[b], out_hbm.at[pl.ds(0, CHUNK)], osems[b]).wait()

    for b in range(NBUF):
        start_load(b, b)

    @pl.loop(0, NCHUNK // NBUF)
    def _step(s):
        for b in range(NBUF):
            g = s * NBUF + b
            wait_load(b)

            @pl.when(s > 0)
            def _():
                wait_store(b)

            tb = tbufs[b]
            pb = pbufs[b]
            ob = obufs[b]

            @plsc.parallel_loop(0, CHUNK // LANES, unroll=16)
            def _vec(i):
                sl = pl.ds(i * LANES, LANES)
                t = tb[sl]
                p = pb[sl]
                tn = jnp.minimum(t * C_T, np.float32(THETA_RES - 1))
                pn = jnp.minimum(p * C_P, np.float32(PHI_RES - 1))
                idx = tn.astype(jnp.int32) * PHI_RES + pn.astype(jnp.int32)
                ob[sl] = plsc.load_gather(table, [idx])

            start_store(g, b)

            @pl.when(g + NBUF < NCHUNK)
            def _():
                start_load(g + NBUF, b)

    for b in range(NBUF):
        wait_store(b)


@jax.jit
def _sc_call(theta, phi, gridf):
    mesh = plsc.VectorSubcoreMesh(core_axis_name="c", subcore_axis_name="s")
    return pl.kernel(
        _sc_body,
        out_type=jax.ShapeDtypeStruct((N,), jnp.float32),
        mesh=mesh,
        scratch_types=(
            [pltpu.VMEM((CHUNK,), jnp.float32)] * 12
            + [pltpu.VMEM((THETA_RES * PHI_RES,), jnp.float32)] * 2
            + [pltpu.SemaphoreType.DMA] * 8
        ),
        compiler_params=pltpu.CompilerParams(needs_layout_passes=False),
    )(theta, phi, gridf)


def kernel(theta, phi, grid):
    return _sc_call(theta, phi, grid.reshape(-1))


# speedup vs baseline: 2.0280x; 1.0564x over previous
import functools

import jax
import jax.numpy as jnp
import numpy as np
from jax import lax
from jax.experimental import pallas as pl
from jax.experimental.pallas import tpu as pltpu
from jax.experimental.pallas import tpu_sc as plsc

THETA_RES = 16
PHI_RES = 8
MAX_SCALE_LOG = 0.3
N = 16777216

NC = 2
NS = 16
NW = NC * NS
B_PER_W = N // NW
CHUNK = 8192
NCHUNK = B_PER_W // CHUNK
LANES = 16
NBUF = 4

TAB_STRIDE = PHI_RES + 1
TAB_ROWS = THETA_RES + 1
TAB_PAD = 160

C_T = np.float32(np.float64(THETA_RES) / (2.0 * np.pi))
C_P = np.float32(np.float64(PHI_RES) / np.pi)


def _sc_body(theta_hbm, phi_hbm, grid_hbm, out_hbm,
             tbuf0, tbuf1, tbuf2, tbuf3,
             pbuf0, pbuf1, pbuf2, pbuf3,
             obuf0, obuf1, obuf2, obuf3,
             gbuf, table,
             lsem0, lsem1, lsem2, lsem3,
             osem0, osem1, osem2, osem3):
    wid = lax.axis_index("s") * NC + lax.axis_index("c")
    base = wid * B_PER_W
    tbufs = (tbuf0, tbuf1, tbuf2, tbuf3)
    pbufs = (pbuf0, pbuf1, pbuf2, pbuf3)
    obufs = (obuf0, obuf1, obuf2, obuf3)
    lsems = (lsem0, lsem1, lsem2, lsem3)
    osems = (osem0, osem1, osem2, osem3)

    pltpu.sync_copy(grid_hbm, gbuf)

    @pl.loop(0, TAB_PAD // LANES)
    def _table(v):
        g = gbuf[pl.ds(v * LANES, LANES)]
        table[pl.ds(v * LANES, LANES)] = jnp.exp(
            jnp.clip(g, -MAX_SCALE_LOG, MAX_SCALE_LOG))

    def start_load(g, b):
        off = base + g * CHUNK
        pltpu.async_copy(theta_hbm.at[pl.ds(off, CHUNK)], tbufs[b], lsems[b])
        pltpu.async_copy(phi_hbm.at[pl.ds(off, CHUNK)], pbufs[b], lsems[b])

    def wait_load(b):
        pltpu.make_async_copy(
            theta_hbm.at[pl.ds(0, CHUNK)], tbufs[b], lsems[b]).wait()
        pltpu.make_async_copy(
            phi_hbm.at[pl.ds(0, CHUNK)], pbufs[b], lsems[b]).wait()

    def start_store(g, b):
        off = base + g * CHUNK
        pltpu.async_copy(obufs[b], out_hbm.at[pl.ds(off, CHUNK)], osems[b])

    def wait_store(b):
        pltpu.make_async_copy(
            obufs[b], out_hbm.at[pl.ds(0, CHUNK)], osems[b]).wait()

    for b in range(NBUF):
        start_load(b, b)

    @pl.loop(0, NCHUNK // NBUF)
    def _step(s):
        for b in range(NBUF):
            g = s * NBUF + b
            wait_load(b)

            @pl.when(s > 0)
            def _():
                wait_store(b)

            tb = tbufs[b]
            pb = pbufs[b]
            ob = obufs[b]

            @plsc.parallel_loop(0, CHUNK // LANES, unroll=16)
            def _vec(i):
                sl = pl.ds(i * LANES, LANES)
                ti = (tb[sl] * C_T).astype(jnp.int32)
                pi = (pb[sl] * C_P).astype(jnp.int32)
                idx = ti * TAB_STRIDE + pi
                ob[sl] = plsc.load_gather(table, [idx])

            start_store(g, b)

            @pl.when(g + NBUF < NCHUNK)
            def _():
                start_load(g + NBUF, b)

    for b in range(NBUF):
        wait_store(b)


@jax.jit
def _sc_call(theta, phi, gridp):
    mesh = plsc.VectorSubcoreMesh(core_axis_name="c", subcore_axis_name="s")
    return pl.kernel(
        _sc_body,
        out_type=jax.ShapeDtypeStruct((N,), jnp.float32),
        mesh=mesh,
        scratch_types=(
            [pltpu.VMEM((CHUNK,), jnp.float32)] * 12
            + [pltpu.VMEM((TAB_PAD,), jnp.float32)] * 2
            + [pltpu.SemaphoreType.DMA] * 8
        ),
        compiler_params=pltpu.CompilerParams(needs_layout_passes=False),
    )(theta, phi, gridp)


def kernel(theta, phi, grid):
    g = jnp.concatenate([grid, grid[:, -1:]], axis=1)
    g = jnp.concatenate([g, g[-1:, :]], axis=0)
    gridp = jnp.pad(g.reshape(-1), (0, TAB_PAD - TAB_ROWS * TAB_STRIDE))
    return _sc_call(theta, phi, gridp)
